# baseline (speedup 1.0000x reference)
import functools

import jax
import jax.numpy as jnp
from jax import lax
from jax.experimental import pallas as pl
from jax.experimental.pallas import tpu as pltpu
from jax.experimental.pallas import tpu_sc as plsc

_N = 4
_K = 2

_mesh = plsc.ScalarSubcoreMesh(axis_name="c", num_cores=1)


@functools.partial(
    pl.kernel,
    mesh=_mesh,
    out_type=jax.ShapeDtypeStruct((_N,), jnp.float32),
    scratch_types=[
        pltpu.SMEM((_N,), jnp.float32),
        pltpu.SMEM((_N,), jnp.float32),
    ],
    compiler_params=pltpu.CompilerParams(needs_layout_passes=False),
)
def _gate_topk_scs(gate_hbm, out_hbm, g_s, o_s):
    pltpu.sync_copy(gate_hbm, g_s)
    x = [g_s[i] for i in range(_N)]
    m = jnp.maximum(jnp.maximum(x[0], x[1]), jnp.maximum(x[2], x[3]))

    def _exp(z):
        t = jnp.maximum(z * jnp.float32(1.4426950408889634), jnp.float32(-28.0))
        ti = t.astype(jnp.int32)
        fi = ti.astype(jnp.float32)
        ti = ti - (fi > t).astype(jnp.int32)
        f = t - ti.astype(jnp.float32)
        p = jnp.float32(1.3697664475809267e-2)
        p = p * f + jnp.float32(5.1690358205939469e-2)
        p = p * f + jnp.float32(2.4163844572498163e-1)
        p = p * f + jnp.float32(6.9296612266139567e-1)
        p = p * f + jnp.float32(1.0000026977044459e0)
        pow2i = lax.shift_left(jnp.int32(1), ti + jnp.int32(29)).astype(
            jnp.float32) * jnp.float32(2.0 ** -29)
        return p * pow2i

    e = [_exp(xi - m) for xi in x]
    s = e[0] + e[1] + e[2] + e[3]
    y = jnp.float32(0.9) - jnp.float32(0.2) * s
    for _ in range(5):
        y = y * (jnp.float32(2.0) - s * y)
    inv = y
    for i in range(_N):
        rank = 0
        for j in range(_N):
            if j == i:
                continue
            ahead = (x[j] > x[i]) | ((x[j] == x[i]) & (j < i))
            rank = rank + ahead.astype(jnp.int32)
        o_s[i] = jnp.where(rank < _K, e[i] * inv, 0.0)
    pltpu.sync_copy(o_s, out_hbm)


def kernel(gate):
    return _gate_topk_scs(gate)
